# baseline (speedup 1.0000x reference)
import functools

import jax
import jax.numpy as jnp
from jax import lax
from jax.experimental import pallas as pl
from jax.experimental.pallas import tpu as pltpu
from jax.experimental.pallas import tpu_sc as plsc

_LANES = 16
_UNIT = 1024
_TOKW = 256
_NW = 32
_WINFLIGHT = 8


@functools.partial(jax.jit, static_argnums=(2,))
def _sc_override_lookup(tok_t, table_flat, d):
    h, nb = tok_t.shape
    v = table_flat.shape[0] // d
    nq = d // _LANES
    h2 = h // 2
    mesh = plsc.VectorSubcoreMesh(core_axis_name="c", subcore_axis_name="s")

    @functools.partial(
        pl.kernel,
        out_type=jax.ShapeDtypeStruct((h, d, nb), jnp.float32),
        mesh=mesh,
        compiler_params=pltpu.CompilerParams(needs_layout_passes=False),
        scratch_types=[
            pltpu.VMEM((v * d,), jnp.float32),
            pltpu.VMEM((h, _TOKW), jnp.int32),
            pltpu.VMEM((d, _UNIT), jnp.float32),
            pltpu.SemaphoreType.DMA,
        ],
    )
    def body(tok_hbm, table_hbm, out_hbm, table_v, tok_v, cbuf, sem_out):
        wid = lax.axis_index("s") * 2 + lax.axis_index("c")
        bu = lax.shift_right_logical(wid, 1)
        hhalf = lax.bitwise_and(wid, 1)
        b0 = bu * _UNIT
        hbase = hhalf * h2
        pltpu.sync_copy(table_hbm, table_v)

        def fill_pattern():
            def fbody(j, c2):
                off = pl.multiple_of(j * _LANES, _LANES)
                for q in range(nq):
                    t16 = table_v[pl.ds(q * _LANES, _LANES)]
                    for l in range(_LANES):
                        val = jnp.broadcast_to(t16[l], (_LANES,))
                        cbuf[q * _LANES + l, pl.ds(off, _LANES)] = val
                return c2

            lax.fori_loop(0, _UNIT // _LANES, fbody, 0)

        fill_pattern()

        def wait_one():
            pltpu.make_async_copy(
                cbuf, out_hbm.at[0, :, pl.ds(0, _UNIT)], sem_out
            ).wait()

        def drain(outstanding):
            def wbody(o):
                wait_one()
                return o - 1

            return lax.while_loop(lambda o: o > 0, wbody, outstanding)

        lane = lax.iota(jnp.int32, _LANES)
        jpg = _TOKW // _LANES

        def load_tok(sub):
            pltpu.sync_copy(
                tok_hbm.at[:, pl.ds(b0 + sub * _TOKW, _TOKW)], tok_v
            )

        m = jnp.zeros((_LANES,), jnp.int32)
        for sub in range(_UNIT // _TOKW):
            load_tok(sub)

            def scan(g, mm):
                hh = hbase + lax.shift_right_logical(g, 4)
                j = lax.bitwise_and(g, jpg - 1)
                toff = pl.multiple_of(j * _LANES, _LANES)
                return jnp.minimum(mm, tok_v[hh, pl.ds(toff, _LANES)])

            m = lax.fori_loop(0, h2 * jpg, scan, m, unroll=8)
        any_neg = jnp.min(m) < 0

        def fast(o):
            def hbody(i, o2):
                o2 = lax.cond(
                    o2 >= _WINFLIGHT,
                    lambda x: drain(x - (_WINFLIGHT - 1)) + (_WINFLIGHT - 1),
                    lambda x: x,
                    o2,
                )
                pltpu.make_async_copy(
                    cbuf, out_hbm.at[hbase + i, :, pl.ds(b0, _UNIT)], sem_out
                ).start()
                return o2 + 1

            return lax.fori_loop(0, h2, hbody, o)

        def slow(o):
            o = drain(o)
            for sub in range(_UNIT // _TOKW):
                load_tok(sub)

                def hbody(i, c2):
                    habs = hbase + i

                    def gbody(g, c3):
                        toff = pl.multiple_of(g * _LANES, _LANES)
                        t = tok_v[habs, pl.ds(toff, _LANES)]
                        r = jnp.minimum(jnp.maximum(-t, 0), v - 1) * d
                        for l in range(_LANES):
                            rl = r[l]
                            col = jnp.broadcast_to(
                                (g * _LANES + l).astype(jnp.int32), (_LANES,)
                            )
                            for q in range(nq):
                                vals = table_v[pl.ds(rl + q * _LANES, _LANES)]
                                plsc.store_scatter(
                                    cbuf, [q * _LANES + lane, col], vals
                                )
                        return c3

                    lax.fori_loop(0, jpg, gbody, 0)
                    cp = pltpu.make_async_copy(
                        cbuf.at[:, pl.ds(0, _TOKW)],
                        out_hbm.at[habs, :, pl.ds(b0 + sub * _TOKW, _TOKW)],
                        sem_out,
                    )
                    cp.start()
                    cp.wait()
                    return c2

                lax.fori_loop(0, h2, hbody, 0)
            fill_pattern()
            return o

        outstanding = lax.cond(any_neg, slow, fast, jnp.int32(0))
        drain(outstanding)

    return body(tok_t, table_flat)


def kernel(tokens, wte, wte_override):
    del wte
    b, h = tokens.shape
    v, d = wte_override.shape
    assert b % (_UNIT * _NW // 2) == 0 and d % _LANES == 0
    assert h % 2 == 0 and _TOKW == 16 * _LANES
    out = _sc_override_lookup(tokens.T, wte_override.reshape(v * d), d)
    return jnp.transpose(out, (2, 0, 1))


# speedup vs baseline: 1.0151x; 1.0151x over previous
import functools

import jax
import jax.numpy as jnp
from jax import lax
from jax.experimental import pallas as pl
from jax.experimental.pallas import tpu as pltpu
from jax.experimental.pallas import tpu_sc as plsc

_LANES = 16
_UNIT = 256
_NW = 32
_WINFLIGHT = 16


@functools.partial(jax.jit, static_argnums=(2,))
def _sc_override_lookup(tok_t, table_flat, d):
    h, nb = tok_t.shape
    v = table_flat.shape[0] // d
    units_per_w = nb // (_UNIT * _NW)
    nq = d // _LANES
    mesh = plsc.VectorSubcoreMesh(core_axis_name="c", subcore_axis_name="s")

    @functools.partial(
        pl.kernel,
        out_type=jax.ShapeDtypeStruct((h, d, nb), jnp.float32),
        mesh=mesh,
        compiler_params=pltpu.CompilerParams(needs_layout_passes=False),
        scratch_types=[
            pltpu.VMEM((v * d,), jnp.float32),
            pltpu.VMEM((h, _UNIT), jnp.int32),
            pltpu.VMEM((d, _UNIT), jnp.float32),
            pltpu.VMEM((d, _UNIT), jnp.float32),
            pltpu.SemaphoreType.DMA,
        ],
    )
    def body(tok_hbm, table_hbm, out_hbm, table_v, tok_v, cbuf, sbuf, sem_out):
        wid = lax.axis_index("s") * 2 + lax.axis_index("c")
        pltpu.sync_copy(table_hbm, table_v)

        for q in range(nq):
            t16 = table_v[pl.ds(q * _LANES, _LANES)]
            for l in range(_LANES):
                val = jnp.broadcast_to(t16[l], (_LANES,))
                dd = q * _LANES + l
                for j in range(_UNIT // _LANES):
                    cbuf[dd, pl.ds(j * _LANES, _LANES)] = val

        def wait_one():
            pltpu.make_async_copy(
                cbuf, out_hbm.at[0, :, pl.ds(0, _UNIT)], sem_out
            ).wait()

        def drain(outstanding):
            def wbody(o):
                wait_one()
                return o - 1

            return lax.while_loop(lambda o: o > 0, wbody, outstanding)

        lane = lax.iota(jnp.int32, _LANES)

        jpg = _UNIT // _LANES

        def unit(u, outstanding):
            b0 = u * _UNIT
            pltpu.sync_copy(
                tok_hbm.at[:, pl.ds(pl.multiple_of(b0, _UNIT), _UNIT)], tok_v
            )

            def scan(g, m):
                hh = lax.shift_right_logical(g, 4)
                j = lax.bitwise_and(g, jpg - 1)
                toff = pl.multiple_of(j * _LANES, _LANES)
                return jnp.minimum(m, tok_v[hh, pl.ds(toff, _LANES)])

            m = lax.fori_loop(
                0, h * jpg, scan,
                jnp.zeros((_LANES,), jnp.int32), unroll=8,
            )
            any_neg = jnp.min(m) < 0

            def fast(o):
                def hbody(hh, o2):
                    o2 = lax.cond(
                        o2 >= _WINFLIGHT,
                        lambda x: drain(x - (_WINFLIGHT - 1)) + (_WINFLIGHT - 1),
                        lambda x: x,
                        o2,
                    )
                    pltpu.make_async_copy(
                        cbuf, out_hbm.at[hh, :, pl.ds(b0, _UNIT)], sem_out
                    ).start()
                    return o2 + 1

                return lax.fori_loop(0, h, hbody, o)

            def slow(o):
                o = drain(o)

                def hbody(hh, c2):
                    def gbody(g, c3):
                        t = tok_v[hh, pl.ds(pl.multiple_of(g * _LANES, _LANES), _LANES)]
                        r = jnp.minimum(jnp.maximum(-t, 0), v - 1) * d
                        for l in range(_LANES):
                            rl = r[l]
                            col = jnp.broadcast_to(
                                jnp.int32(g * _LANES + l), (_LANES,)
                            )
                            for q in range(nq):
                                vals = table_v[pl.ds(rl + q * _LANES, _LANES)]
                                plsc.store_scatter(
                                    sbuf, [q * _LANES + lane, col], vals
                                )
                        return c3

                    lax.fori_loop(0, _UNIT // _LANES, gbody, 0)
                    cp = pltpu.make_async_copy(
                        sbuf, out_hbm.at[hh, :, pl.ds(b0, _UNIT)], sem_out
                    )
                    cp.start()
                    cp.wait()
                    return c2

                lax.fori_loop(0, h, hbody, 0)
                return o

            return lax.cond(any_neg, slow, fast, outstanding)

        outstanding = jnp.int32(0)
        for i in range(units_per_w):
            outstanding = unit(wid * units_per_w + i, outstanding)
        drain(outstanding)

    return body(tok_t, table_flat)


def kernel(tokens, wte, wte_override):
    del wte
    b, h = tokens.shape
    v, d = wte_override.shape
    assert b % (_UNIT * _NW) == 0 and d % _LANES == 0 and _UNIT == 16 * _LANES
    out = _sc_override_lookup(tokens.T, wte_override.reshape(v * d), d)
    return jnp.transpose(out, (2, 0, 1))
